# baseline (speedup 1.0000x reference)
import functools

import jax
import jax.numpy as jnp
from jax import lax
from jax.experimental import pallas as pl
from jax.experimental.pallas import tpu as pltpu
from jax.experimental.pallas import tpu_sc as plsc

_B = 1024
_S = 200
_DIM = 128
_HALF = _DIM // 2
_BASE = 10000

_NC = 2
_NS = 16
_NW = _NC * _NS
_BPW = _B // _NW
_G0 = 96
_G1 = _S - _G0
_NSLOT = 3
_NIDX = 4


def _sincos():
    inv_freq = 1.0 / (_BASE ** (jnp.arange(0, _HALF, dtype=jnp.float32) / _HALF))
    angles = jnp.arange(_S, dtype=jnp.float32)[:, None] * inv_freq[None, :]
    return jnp.sin(angles), jnp.cos(angles)


def _body(ids0_ref, ids1_ref, table_ref, sin_ref, cos_ref, out_ref,
          idx0_v, idx1_v, rows_v, sin_v, cos_v,
          gsem0, gsem1, gsem2, wsem0, wsem1, wsem2, isem):
    wid = lax.axis_index("s") * _NC + lax.axis_index("c")
    base = wid * _BPW

    pltpu.sync_copy(sin_ref, sin_v)
    pltpu.sync_copy(cos_ref, cos_v)

    gsem = (gsem0, gsem1, gsem2)
    wsem = (wsem0, wsem1, wsem2)

    def start_ids(u):
        r = u % _NIDX
        return [
            pltpu.async_copy(ids0_ref.at[base + u], idx0_v.at[r], isem),
            pltpu.async_copy(ids1_ref.at[base + u], idx1_v.at[r], isem),
        ]

    def start_gather(u):
        r = u % _NIDX
        s = u % _NSLOT
        return [
            pltpu.async_copy(table_ref.at[idx0_v.at[r]],
                             rows_v.at[s, pl.ds(0, _G0)], gsem[s]),
            pltpu.async_copy(table_ref.at[idx1_v.at[r]],
                             rows_v.at[s, pl.ds(_G0, _G1)], gsem[s]),
        ]

    def compute(u):
        s = u % _NSLOT

        @plsc.parallel_loop(0, _S, step=1, unroll=2)
        def row_body(i):
            for j in range(_HALF // 16):
                lo = pl.ds(j * 16, 16)
                hi = pl.ds(_HALF + j * 16, 16)
                cosv = cos_v[i, lo]
                sinv = sin_v[i, lo]
                t1 = rows_v[s, i, lo]
                t2 = rows_v[s, i, hi]
                rows_v[s, i, lo] = t1 * cosv - t2 * sinv
                rows_v[s, i, hi] = t1 * sinv + t2 * cosv

    def start_write(u):
        s = u % _NSLOT
        return pltpu.async_copy(rows_v.at[s], out_ref.at[base + u], wsem[s])

    ipend = [start_ids(u) for u in range(min(_NIDX - 1, _BPW))]
    ipend += [None] * (_BPW - len(ipend))
    wpend = [None] * _NSLOT
    gpend = [None] * _NSLOT
    for u in range(min(2, _BPW)):
        for c in ipend[u]:
            c.wait()
        ipend[u] = None
        gpend[u % _NSLOT] = start_gather(u)
    for u in range(_BPW):
        s = u % _NSLOT
        if u + 2 < _BPW:
            sn = (u + 2) % _NSLOT
            if wpend[sn] is not None:
                wpend[sn].wait()
                wpend[sn] = None
            for c in ipend[u + 2]:
                c.wait()
            ipend[u + 2] = None
            gpend[sn] = start_gather(u + 2)
        if u + 3 < _BPW:
            ipend[u + 3] = start_ids(u + 3)
        for cp in gpend[s]:
            cp.wait()
        gpend[s] = None
        compute(u)
        wpend[s] = start_write(u)
    for w in wpend:
        if w is not None:
            w.wait()


@jax.jit
def _run(ids0, ids1, table, sin, cos):
    mesh = plsc.VectorSubcoreMesh(core_axis_name="c", subcore_axis_name="s",
                                  num_cores=_NC, num_subcores=_NS)
    f = pl.kernel(
        _body,
        out_type=jax.ShapeDtypeStruct((_B, _S, _DIM), jnp.float32),
        mesh=mesh,
        scratch_types=[
            pltpu.VMEM((_NIDX, _G0), jnp.int32),
            pltpu.VMEM((_NIDX, _G1), jnp.int32),
            pltpu.VMEM((_NSLOT, _S, _DIM), jnp.float32),
            pltpu.VMEM((_S, _HALF), jnp.float32),
            pltpu.VMEM((_S, _HALF), jnp.float32),
            pltpu.SemaphoreType.DMA,
            pltpu.SemaphoreType.DMA,
            pltpu.SemaphoreType.DMA,
            pltpu.SemaphoreType.DMA,
            pltpu.SemaphoreType.DMA,
            pltpu.SemaphoreType.DMA,
            pltpu.SemaphoreType.DMA,
        ],
    )
    return f(ids0, ids1, table, sin, cos)


def kernel(ids, table):
    sin, cos = _sincos()
    ids0 = ids[:, :_G0]
    ids1 = ids[:, _G0:]
    return _run(ids0, ids1, table, sin, cos)


# speedup vs baseline: 1.0543x; 1.0543x over previous
import functools

import jax
import jax.numpy as jnp
from jax import lax
from jax.experimental import pallas as pl
from jax.experimental.pallas import tpu as pltpu
from jax.experimental.pallas import tpu_sc as plsc

_B = 1024
_S = 200
_DIM = 128
_HALF = _DIM // 2
_BASE = 10000

_NC = 2
_NS = 16
_NW = _NC * _NS
_BPW = _B // _NW
_GS = (64, 64, 72)
_POS = (0, 64, 128)
_NWIN = len(_GS)
_TPW = (_BPW // 2) * _NWIN


def _sincos():
    inv_freq = 1.0 / (_BASE ** (jnp.arange(0, _HALF, dtype=jnp.float32) / _HALF))
    angles = jnp.arange(_S, dtype=jnp.float32)[:, None] * inv_freq[None, :]
    return jnp.sin(angles), jnp.cos(angles)


def _body(ids0_ref, ids1_ref, ids2_ref, table_ref, sin_ref, cos_ref, out_ref,
          idx0_v, idx1_v, idx2_v, rows0_v, rows1_v, rows2_v, sin_v, cos_v,
          gsem0, gsem1, gsem2, wsem0, wsem1, wsem2):
    wid = lax.axis_index("s") * _NC + lax.axis_index("c")
    base = wid * _BPW

    pltpu.sync_copy(sin_ref, sin_v)
    pltpu.sync_copy(cos_ref, cos_v)
    pltpu.sync_copy(ids0_ref.at[pl.ds(base, _BPW)], idx0_v)
    pltpu.sync_copy(ids1_ref.at[pl.ds(base, _BPW)], idx1_v)
    pltpu.sync_copy(ids2_ref.at[pl.ds(base, _BPW)], idx2_v)

    gsem = (gsem0, gsem1, gsem2)
    wsem = (wsem0, wsem1, wsem2)
    cfg = (
        (idx0_v, rows0_v, _GS[0], _POS[0]),
        (idx1_v, rows1_v, _GS[1], _POS[1]),
        (idx2_v, rows2_v, _GS[2], _POS[2]),
    )

    def local_batches(t):
        q = t // _NWIN
        return 2 * q, 2 * q + 1

    def start_gather(t):
        idx_v, rows_v, g, _ = cfg[t % _NWIN]
        k0, k1 = local_batches(t)
        return [
            pltpu.async_copy(table_ref.at[idx_v.at[k]],
                             rows_v.at[pl.ds(c * g, g)], gsem[t % _NWIN])
            for c, k in enumerate((k0, k1))
        ]

    def compute(t):
        _, rows_v, g, pos0 = cfg[t % _NWIN]

        @plsc.parallel_loop(0, g, step=1, unroll=2)
        def row_body(i):
            for j in range(_HALF // 16):
                lo = pl.ds(j * 16, 16)
                hi = pl.ds(_HALF + j * 16, 16)
                cosv = cos_v[pos0 + i, lo]
                sinv = sin_v[pos0 + i, lo]
                for u in range(2):
                    r = u * g + i
                    t1 = rows_v[r, lo]
                    t2 = rows_v[r, hi]
                    rows_v[r, lo] = t1 * cosv - t2 * sinv
                    rows_v[r, hi] = t1 * sinv + t2 * cosv

    def start_write(t):
        _, rows_v, g, pos0 = cfg[t % _NWIN]
        k0, k1 = local_batches(t)
        sl = pl.ds(pos0, g)
        return [
            pltpu.async_copy(rows_v.at[pl.ds(0, g)],
                             out_ref.at[base + k0, sl], wsem[t % _NWIN]),
            pltpu.async_copy(rows_v.at[pl.ds(g, g)],
                             out_ref.at[base + k1, sl], wsem[t % _NWIN]),
        ]

    wpend = [None] * _NWIN
    gpend = [None] * _NWIN
    for t in range(min(2, _TPW)):
        gpend[t % _NWIN] = start_gather(t)
    for t in range(_TPW):
        s = t % _NWIN
        if t + 2 < _TPW:
            sn = (t + 2) % _NWIN
            if wpend[sn] is not None:
                for w in wpend[sn]:
                    w.wait()
                wpend[sn] = None
            gpend[sn] = start_gather(t + 2)
        for cp in gpend[s]:
            cp.wait()
        gpend[s] = None
        compute(t)
        wpend[s] = start_write(t)
    for ws in wpend:
        if ws is not None:
            for w in ws:
                w.wait()


@jax.jit
def _run(ids0, ids1, ids2, table, sin, cos):
    mesh = plsc.VectorSubcoreMesh(core_axis_name="c", subcore_axis_name="s",
                                  num_cores=_NC, num_subcores=_NS)
    f = pl.kernel(
        _body,
        out_type=jax.ShapeDtypeStruct((_B, _S, _DIM), jnp.float32),
        mesh=mesh,
        scratch_types=[
            pltpu.VMEM((_BPW, _GS[0]), jnp.int32),
            pltpu.VMEM((_BPW, _GS[1]), jnp.int32),
            pltpu.VMEM((_BPW, _GS[2]), jnp.int32),
            pltpu.VMEM((2 * _GS[0], _DIM), jnp.float32),
            pltpu.VMEM((2 * _GS[1], _DIM), jnp.float32),
            pltpu.VMEM((2 * _GS[2], _DIM), jnp.float32),
            pltpu.VMEM((_S, _HALF), jnp.float32),
            pltpu.VMEM((_S, _HALF), jnp.float32),
            pltpu.SemaphoreType.DMA,
            pltpu.SemaphoreType.DMA,
            pltpu.SemaphoreType.DMA,
            pltpu.SemaphoreType.DMA,
            pltpu.SemaphoreType.DMA,
            pltpu.SemaphoreType.DMA,
        ],
    )
    return f(ids0, ids1, ids2, table, sin, cos)


def kernel(ids, table):
    sin, cos = _sincos()
    ids0 = ids[:, :_POS[1]]
    ids1 = ids[:, _POS[1]:_POS[2]]
    ids2 = ids[:, _POS[2]:]
    return _run(ids0, ids1, ids2, table, sin, cos)
